# baseline (speedup 1.0000x reference)
import functools

import jax
import jax.numpy as jnp
from jax import lax
from jax.experimental import pallas as pl
from jax.experimental.pallas import tpu as pltpu
from jax.experimental.pallas import tpu_sc as plsc

D_MODEL = 1024
BATCH = 4
SEQ = 2048
N_ROWS = BATCH * SEQ
HALVES = 2
HALF_ROWS = N_ROWS // HALVES
NUM_WORKERS = 32
ROWS_PER_WORKER = HALF_ROWS // NUM_WORKERS
CHUNK = 16
NUM_CHUNKS = ROWS_PER_WORKER // CHUNK
TC_BLOCK = 256
SEQ_BLOCKS = SEQ // TC_BLOCK


def _pos_table(seq_len):
    pos = jnp.arange(0, seq_len, dtype=jnp.float32)[:, None]
    _2i = jnp.arange(0, D_MODEL, 2, dtype=jnp.float32)
    angle = pos / jnp.power(10000.0, _2i / D_MODEL)
    table = jnp.zeros((seq_len, D_MODEL), dtype=jnp.float32)
    table = table.at[:, 0::2].set(jnp.sin(angle))
    table = table.at[:, 1::2].set(jnp.cos(angle))
    return table


_MESH = plsc.VectorSubcoreMesh(core_axis_name="c", subcore_axis_name="s")


@functools.partial(
    pl.kernel,
    out_type=jax.ShapeDtypeStruct((HALF_ROWS, D_MODEL), jnp.float32),
    mesh=_MESH,
    scratch_types=[
        pltpu.VMEM((NUM_CHUNKS, CHUNK), jnp.int32),
        pltpu.VMEM((CHUNK, D_MODEL), jnp.float32),
        pltpu.VMEM((CHUNK, D_MODEL), jnp.float32),
        pltpu.SemaphoreType.DMA,
        pltpu.SemaphoreType.DMA,
        pltpu.SemaphoreType.DMA,
        pltpu.SemaphoreType.DMA,
    ],
)
def _gather_half(tok_hbm, idx_hbm, out_hbm,
                 idx_v, emb0, emb1, sg0, sg1, sw0, sw1):
    wid = lax.axis_index("s") * 2 + lax.axis_index("c")
    base = wid * ROWS_PER_WORKER

    pltpu.sync_copy(idx_hbm.at[wid], idx_v)
    pltpu.async_copy(tok_hbm.at[idx_v.at[0]], emb0, sg0)

    def out_slice(c):
        return out_hbm.at[pl.ds(base + c * CHUNK, CHUNK)]

    @pl.loop(0, NUM_CHUNKS, step=2)
    def _pipe(c):
        @pl.when(c > 0)
        def _():
            pltpu.make_async_copy(emb1, out_slice(c - 1), sw1).wait()
        pltpu.async_copy(tok_hbm.at[idx_v.at[c + 1]], emb1, sg1)
        pltpu.make_async_copy(tok_hbm.at[idx_v.at[c]], emb0, sg0).wait()
        pltpu.async_copy(emb0, out_slice(c), sw0)

        @pl.when(c + 2 < NUM_CHUNKS)
        def _():
            pltpu.make_async_copy(emb0, out_slice(c), sw0).wait()
            pltpu.async_copy(tok_hbm.at[idx_v.at[c + 2]], emb0, sg0)
        pltpu.make_async_copy(tok_hbm.at[idx_v.at[c + 1]], emb1, sg1).wait()
        pltpu.async_copy(emb1, out_slice(c + 1), sw1)

    pltpu.make_async_copy(emb0, out_slice(NUM_CHUNKS - 2), sw0).wait()
    pltpu.make_async_copy(emb1, out_slice(NUM_CHUNKS - 1), sw1).wait()


def _add_body(emb0_ref, emb1_ref, pos_ref, out_ref):
    h = pl.program_id(0)

    @pl.when(h == 0)
    def _():
        out_ref[...] = emb0_ref[...] + pos_ref[...]

    @pl.when(h == 1)
    def _():
        out_ref[...] = emb1_ref[...] + pos_ref[...]


def _add_pos(emb_h0, emb_h1, pos):
    bph = HALF_ROWS // SEQ
    return pl.pallas_call(
        _add_body,
        grid=(HALVES, SEQ_BLOCKS, bph),
        in_specs=[
            pl.BlockSpec((TC_BLOCK, D_MODEL),
                         lambda h, i, j: ((1 - h) * (j * SEQ_BLOCKS + i), 0)),
            pl.BlockSpec((TC_BLOCK, D_MODEL),
                         lambda h, i, j: (h * (j * SEQ_BLOCKS + i), 0)),
            pl.BlockSpec((TC_BLOCK, D_MODEL), lambda h, i, j: (i, 0)),
        ],
        out_specs=pl.BlockSpec(
            (TC_BLOCK, D_MODEL),
            lambda h, i, j: ((h * bph + j) * SEQ_BLOCKS + i, 0)),
        out_shape=jax.ShapeDtypeStruct((N_ROWS, D_MODEL), jnp.float32),
    )(emb_h0, emb_h1, pos)


def kernel(x, tok_emb):
    idx = (x.astype(jnp.int32)
           .reshape(HALVES, NUM_WORKERS, NUM_CHUNKS, CHUNK))
    pos = _pos_table(SEQ)
    emb_h0 = _gather_half(tok_emb, idx[0])
    emb_h1 = _gather_half(tok_emb, idx[1])
    out = _add_pos(emb_h0, emb_h1, pos)
    return out.reshape(BATCH, SEQ, D_MODEL)


# speedup vs baseline: 1.0047x; 1.0047x over previous
import functools

import jax
import jax.numpy as jnp
from jax import lax
from jax.experimental import pallas as pl
from jax.experimental.pallas import tpu as pltpu
from jax.experimental.pallas import tpu_sc as plsc

D_MODEL = 1024
BATCH = 4
SEQ = 2048
N_ROWS = BATCH * SEQ
HALVES = 2
HALF_ROWS = N_ROWS // HALVES
NUM_WORKERS = 32
ROWS_PER_WORKER = HALF_ROWS // NUM_WORKERS
CHUNK = 16
NUM_CHUNKS = ROWS_PER_WORKER // CHUNK
TC_BLOCK = 256
SEQ_BLOCKS = SEQ // TC_BLOCK


def _pos_table(seq_len):
    pos = jnp.arange(0, seq_len, dtype=jnp.float32)[:, None]
    _2i = jnp.arange(0, D_MODEL, 2, dtype=jnp.float32)
    angle = pos / jnp.power(10000.0, _2i / D_MODEL)
    table = jnp.zeros((seq_len, D_MODEL), dtype=jnp.float32)
    table = table.at[:, 0::2].set(jnp.sin(angle))
    table = table.at[:, 1::2].set(jnp.cos(angle))
    return table


_MESH = plsc.VectorSubcoreMesh(core_axis_name="c", subcore_axis_name="s")


@functools.partial(
    pl.kernel,
    out_type=jax.ShapeDtypeStruct((HALF_ROWS, D_MODEL), jnp.float32),
    mesh=_MESH,
    scratch_types=[
        pltpu.VMEM((NUM_CHUNKS, CHUNK), jnp.int32),
        pltpu.VMEM((CHUNK, D_MODEL), jnp.float32),
        pltpu.VMEM((CHUNK, D_MODEL), jnp.float32),
        pltpu.SemaphoreType.DMA,
        pltpu.SemaphoreType.DMA,
        pltpu.SemaphoreType.DMA,
        pltpu.SemaphoreType.DMA,
    ],
)
def _gather_half(tok_hbm, idx_hbm, out_hbm,
                 idx_v, emb0, emb1, sg0, sg1, sw0, sw1):
    wid = lax.axis_index("s") * 2 + lax.axis_index("c")
    base = wid * ROWS_PER_WORKER

    pltpu.sync_copy(idx_hbm.at[wid], idx_v)
    pltpu.async_copy(tok_hbm.at[idx_v.at[0]], emb0, sg0)

    def out_slice(c):
        return out_hbm.at[pl.ds(base + c * CHUNK, CHUNK)]

    @pl.loop(0, NUM_CHUNKS, step=2)
    def _pipe(c):
        @pl.when(c > 0)
        def _():
            pltpu.make_async_copy(emb1, out_slice(c - 1), sw1).wait()
        pltpu.async_copy(tok_hbm.at[idx_v.at[c + 1]], emb1, sg1)
        pltpu.make_async_copy(tok_hbm.at[idx_v.at[c]], emb0, sg0).wait()
        pltpu.async_copy(emb0, out_slice(c), sw0)

        @pl.when(c + 2 < NUM_CHUNKS)
        def _():
            pltpu.make_async_copy(emb0, out_slice(c), sw0).wait()
            pltpu.async_copy(tok_hbm.at[idx_v.at[c + 2]], emb0, sg0)
        pltpu.make_async_copy(tok_hbm.at[idx_v.at[c + 1]], emb1, sg1).wait()
        pltpu.async_copy(emb1, out_slice(c + 1), sw1)

    pltpu.make_async_copy(emb0, out_slice(NUM_CHUNKS - 2), sw0).wait()
    pltpu.make_async_copy(emb1, out_slice(NUM_CHUNKS - 1), sw1).wait()


_BPH = HALF_ROWS // SEQ


def _add_first_body(emb_ref, pos_ref, out_ref):
    out_ref[...] = emb_ref[...] + pos_ref[...]


def _add_next_body(prev_ref, emb_ref, pos_ref, out_ref):
    del prev_ref
    out_ref[...] = emb_ref[...] + pos_ref[...]


def _add_pos_half(h, emb_h, pos, prev=None):
    emb_spec = pl.BlockSpec((TC_BLOCK, D_MODEL),
                            lambda i, j: (j * SEQ_BLOCKS + i, 0))
    pos_spec = pl.BlockSpec((TC_BLOCK, D_MODEL), lambda i, j: (i, 0))
    out_spec = pl.BlockSpec(
        (TC_BLOCK, D_MODEL),
        lambda i, j, h=h: ((h * _BPH + j) * SEQ_BLOCKS + i, 0))
    out_shape = jax.ShapeDtypeStruct((N_ROWS, D_MODEL), jnp.float32)
    if prev is None:
        return pl.pallas_call(
            _add_first_body,
            grid=(SEQ_BLOCKS, _BPH),
            in_specs=[emb_spec, pos_spec],
            out_specs=out_spec,
            out_shape=out_shape,
        )(emb_h, pos)
    prev_spec = pl.BlockSpec((8, 128), lambda i, j: (0, 0))
    return pl.pallas_call(
        _add_next_body,
        grid=(SEQ_BLOCKS, _BPH),
        in_specs=[prev_spec, emb_spec, pos_spec],
        out_specs=out_spec,
        out_shape=out_shape,
        input_output_aliases={0: 0},
    )(prev, emb_h, pos)


def kernel(x, tok_emb):
    idx = (x.astype(jnp.int32)
           .reshape(HALVES, NUM_WORKERS, NUM_CHUNKS, CHUNK))
    pos = _pos_table(SEQ)
    emb_h0 = _gather_half(tok_emb, idx[0])
    emb_h1 = _gather_half(tok_emb, idx[1])
    out = _add_pos_half(0, emb_h0, pos)
    out = _add_pos_half(1, emb_h1, pos, prev=out)
    return out.reshape(BATCH, SEQ, D_MODEL)


# speedup vs baseline: 1.4125x; 1.4059x over previous
import functools

import numpy as np

import jax
import jax.numpy as jnp
from jax import lax
from jax.experimental import pallas as pl
from jax.experimental.pallas import tpu as pltpu
from jax.experimental.pallas import tpu_sc as plsc

D_MODEL = 1024
BATCH = 4
SEQ = 2048
N_ROWS = BATCH * SEQ
HALVES = 2
HALF_ROWS = N_ROWS // HALVES
NUM_WORKERS = 32
ROWS_PER_WORKER = HALF_ROWS // NUM_WORKERS
CHUNK = 16
NUM_CHUNKS = ROWS_PER_WORKER // CHUNK
TC_BLOCK = 256
SEQ_BLOCKS = SEQ // TC_BLOCK


def _pos_table(seq_len):
    pos = np.arange(0, seq_len, dtype=np.float32)[:, None]
    _2i = np.arange(0, D_MODEL, 2, dtype=np.float32)
    angle = (pos / np.power(10000.0, _2i / D_MODEL)).astype(np.float32)
    table = np.zeros((seq_len, D_MODEL), dtype=np.float32)
    table[:, 0::2] = np.sin(angle)
    table[:, 1::2] = np.cos(angle)
    return table


_POS = _pos_table(SEQ)


_MESH = plsc.VectorSubcoreMesh(core_axis_name="c", subcore_axis_name="s")


@functools.partial(
    pl.kernel,
    out_type=jax.ShapeDtypeStruct((HALF_ROWS, D_MODEL), jnp.float32),
    mesh=_MESH,
    scratch_types=[
        pltpu.VMEM((NUM_CHUNKS, CHUNK), jnp.int32),
        pltpu.VMEM((CHUNK, D_MODEL), jnp.float32),
        pltpu.VMEM((CHUNK, D_MODEL), jnp.float32),
        pltpu.SemaphoreType.DMA,
        pltpu.SemaphoreType.DMA,
        pltpu.SemaphoreType.DMA,
        pltpu.SemaphoreType.DMA,
    ],
)
def _gather_half(tok_hbm, idx_hbm, out_hbm,
                 idx_v, emb0, emb1, sg0, sg1, sw0, sw1):
    wid = lax.axis_index("s") * 2 + lax.axis_index("c")
    base = wid * ROWS_PER_WORKER

    pltpu.sync_copy(idx_hbm.at[wid], idx_v)
    pltpu.async_copy(tok_hbm.at[idx_v.at[0]], emb0, sg0)

    def out_slice(c):
        return out_hbm.at[pl.ds(base + c * CHUNK, CHUNK)]

    @pl.loop(0, NUM_CHUNKS, step=2)
    def _pipe(c):
        @pl.when(c > 0)
        def _():
            pltpu.make_async_copy(emb1, out_slice(c - 1), sw1).wait()
        pltpu.async_copy(tok_hbm.at[idx_v.at[c + 1]], emb1, sg1)
        pltpu.make_async_copy(tok_hbm.at[idx_v.at[c]], emb0, sg0).wait()
        pltpu.async_copy(emb0, out_slice(c), sw0)

        @pl.when(c + 2 < NUM_CHUNKS)
        def _():
            pltpu.make_async_copy(emb0, out_slice(c), sw0).wait()
            pltpu.async_copy(tok_hbm.at[idx_v.at[c + 2]], emb0, sg0)
        pltpu.make_async_copy(tok_hbm.at[idx_v.at[c + 1]], emb1, sg1).wait()
        pltpu.async_copy(emb1, out_slice(c + 1), sw1)

    pltpu.make_async_copy(emb0, out_slice(NUM_CHUNKS - 2), sw0).wait()
    pltpu.make_async_copy(emb1, out_slice(NUM_CHUNKS - 1), sw1).wait()


_BPH = HALF_ROWS // SEQ


def _add_first_body(emb_ref, pos_ref, out_ref):
    out_ref[...] = emb_ref[...] + pos_ref[...]


def _add_next_body(prev_ref, emb_ref, pos_ref, out_ref):
    del prev_ref
    out_ref[...] = emb_ref[...] + pos_ref[...]


def _add_pos_half(h, emb_h, pos, prev=None):
    emb_spec = pl.BlockSpec((TC_BLOCK, D_MODEL),
                            lambda i, j: (j * SEQ_BLOCKS + i, 0))
    pos_spec = pl.BlockSpec((TC_BLOCK, D_MODEL), lambda i, j: (i, 0))
    out_spec = pl.BlockSpec(
        (TC_BLOCK, D_MODEL),
        lambda i, j, h=h: ((h * _BPH + j) * SEQ_BLOCKS + i, 0))
    out_shape = jax.ShapeDtypeStruct((N_ROWS, D_MODEL), jnp.float32)
    if prev is None:
        return pl.pallas_call(
            _add_first_body,
            grid=(SEQ_BLOCKS, _BPH),
            in_specs=[emb_spec, pos_spec],
            out_specs=out_spec,
            out_shape=out_shape,
        )(emb_h, pos)
    prev_spec = pl.BlockSpec((8, 128), lambda i, j: (0, 0))
    return pl.pallas_call(
        _add_next_body,
        grid=(SEQ_BLOCKS, _BPH),
        in_specs=[prev_spec, emb_spec, pos_spec],
        out_specs=out_spec,
        out_shape=out_shape,
        input_output_aliases={0: 0},
    )(prev, emb_h, pos)


def kernel(x, tok_emb):
    idx = (x.astype(jnp.int32)
           .reshape(HALVES, NUM_WORKERS, NUM_CHUNKS, CHUNK))
    pos = jnp.asarray(_POS)
    emb_h0 = _gather_half(tok_emb, idx[0])
    emb_h1 = _gather_half(tok_emb, idx[1])
    out = _add_pos_half(0, emb_h0, pos)
    out = _add_pos_half(1, emb_h1, pos, prev=out)
    return out.reshape(BATCH, SEQ, D_MODEL)
